# Initial kernel scaffold; baseline (speedup 1.0000x reference)
import jax
import jax.numpy as jnp
from jax.experimental import pallas as pl


def kernel(prefix, emb, W1, b1, W2, b2):
    raise NotImplementedError("write your pallas kernel here")


# speedup vs baseline: 1.1061x; 1.1061x over previous
import jax
import jax.numpy as jnp
from jax.experimental import pallas as pl
from jax.experimental.pallas import tpu as pltpu

_TN = 2048


def _body(idx_ref, emb_ref, W1_ref, b1_ref, W2_ref, b2_ref, out_ref,
          h_ref, oh_ref):
    step = pl.program_id(0)

    @pl.when(step == 0)
    def _prologue():
        h_ref[...] = jnp.tanh(
            jnp.dot(emb_ref[...], W1_ref[...],
                    preferred_element_type=jnp.float32) + b1_ref[...])
        T, V = oh_ref.shape
        iota = jax.lax.broadcasted_iota(jnp.int32, (T, V), 1)
        oh_ref[...] = (idx_ref[...] == iota).astype(jnp.float32)

    p = jnp.dot(h_ref[...], W2_ref[...],
                preferred_element_type=jnp.float32) + b2_ref[...]
    out_ref[...] = jnp.dot(oh_ref[...], p,
                           preferred_element_type=jnp.float32)


def kernel(prefix, emb, W1, b1, W2, b2):
    B, L = prefix.shape
    T = B * L
    V, D = emb.shape
    H = W1.shape[1]
    N = W2.shape[1]
    idx = prefix.reshape(T, 1).astype(jnp.int32)
    b1r = b1.reshape(1, H)
    b2r = b2.reshape(1, N)
    grid = N // _TN

    out = pl.pallas_call(
        _body,
        grid=(grid,),
        in_specs=[
            pl.BlockSpec((T, 1), lambda i: (0, 0)),
            pl.BlockSpec((V, D), lambda i: (0, 0)),
            pl.BlockSpec((D, H), lambda i: (0, 0)),
            pl.BlockSpec((1, H), lambda i: (0, 0)),
            pl.BlockSpec((D, _TN), lambda i: (0, i)),
            pl.BlockSpec((1, _TN), lambda i: (0, i)),
        ],
        out_specs=pl.BlockSpec((T, _TN), lambda i: (0, i)),
        out_shape=jax.ShapeDtypeStruct((T, N), jnp.float32),
        scratch_shapes=[
            pltpu.VMEM((V, H), jnp.float32),
            pltpu.VMEM((T, V), jnp.float32),
        ],
    )(idx, emb, W1, b1r, W2, b2r)
    return out.reshape(B, L, N)
